# baseline (speedup 1.0000x reference)
import functools

import jax
import jax.numpy as jnp
from jax import lax
from jax.experimental import pallas as pl
from jax.experimental.pallas import tpu as pltpu
from jax.experimental.pallas import tpu_sc as plsc

B = 16384
EMB = 64
PAIR = 2 * EMB
NC, NS = 2, 16
NW = NC * NS
BPW = B // NW
CH = 128
NCH = BPW // CH
NF = 5


def _sc_gather5(tables2, idx3):
    mesh = plsc.VectorSubcoreMesh(core_axis_name="c", subcore_axis_name="s")
    out_type = tuple(
        jax.ShapeDtypeStruct((B, PAIR), jnp.float32) for _ in range(NF)
    )
    scratch = (
        [pltpu.VMEM((NCH, CH), jnp.int32) for _ in range(NF)]
        + [pltpu.VMEM((CH, PAIR), jnp.float32) for _ in range(NCH)]
        + [pltpu.SemaphoreType.DMA, pltpu.SemaphoreType.DMA]
    )

    @functools.partial(
        pl.kernel, out_type=out_type, mesh=mesh, scratch_types=scratch,
        compiler_params=pltpu.CompilerParams(use_tc_tiling_on_sc=True))
    def k(*refs):
        tbls = refs[0:NF]
        idxr = refs[NF:2 * NF]
        outs = refs[2 * NF:3 * NF]
        idx_v = refs[3 * NF:4 * NF]
        bufs = refs[4 * NF:4 * NF + NCH]
        gsem, wsem = refs[4 * NF + NCH], refs[4 * NF + NCH + 1]

        wid = lax.axis_index("s") * NC + lax.axis_index("c")
        base = wid * BPW

        for f in range(NF):
            pltpu.sync_copy(idxr[f].at[wid], idx_v[f])

        wb = [None] * NCH
        for f in range(NF):
            for j in range(NCH):
                if wb[j] is not None:
                    wb[j].wait()
            gh = [
                pltpu.async_copy(tbls[f].at[idx_v[f].at[j]], bufs[j], gsem)
                for j in range(NCH)
            ]
            for j in range(NCH):
                gh[j].wait()
                wb[j] = pltpu.async_copy(
                    bufs[j], outs[f].at[pl.ds(base + j * CH, CH)], wsem)
        for j in range(NCH):
            wb[j].wait()

    return k(*tables2, *idx3)


def _tc_textproj(text, Wt, bt):
    bm = 2048
    f32 = jnp.float32

    def body(text_ref, wt_ref, bt_ref, out_ref):
        out_ref[...] = lax.dot_general(
            text_ref[...], wt_ref[...], (((1,), (0,)), ((), ())),
            preferred_element_type=f32) + bt_ref[...]

    return pl.pallas_call(
        body, grid=(B // bm,),
        in_specs=[pl.BlockSpec((bm, text.shape[1]), lambda i: (i, 0)),
                  pl.BlockSpec(Wt.shape, lambda i: (0, 0)),
                  pl.BlockSpec(bt.shape, lambda i: (0, 0))],
        out_specs=pl.BlockSpec((bm, EMB), lambda i: (i, 0)),
        out_shape=jax.ShapeDtypeStruct((B, EMB), f32),
    )(text, Wt, bt)


def _tc_towers(p_uid, p_age, p_reg, p_iid, p_cat, par, text_emb,
               Wu1a, Wu1b, Wu1c, bu1, Wu2, bu2, Wu3, bu3,
               Wi1a, Wi1b, Wi1c, bi1, Wi2, bi2, Wi3, bi3):
    bm = 1024
    grid = (B // bm,)
    f32 = jnp.float32

    def dot(a, b):
        return lax.dot_general(a, b, (((1,), (0,)), ((), ())),
                               preferred_element_type=f32)

    def body(uid_ref, age_ref, reg_ref, iid_ref, cat_ref, par_ref, te_ref,
             wu1a_ref, wu1b_ref, wu1c_ref, bu1_ref, wu2_ref, bu2_ref,
             wu3_ref, bu3_ref,
             wi1a_ref, wi1b_ref, wi1c_ref, bi1_ref, wi2_ref, bi2_ref,
             wi3_ref, bi3_ref,
             uout_ref, iout_ref):
        par = par_ref[...]

        def half(pair_ref, k):
            p = par[:, k:k + 1] > 0.5
            return jnp.where(p, pair_ref[:, EMB:], pair_ref[:, :EMB])

        h = (dot(half(uid_ref, 0), wu1a_ref[...])
             + dot(half(age_ref, 1), wu1b_ref[...])
             + dot(half(reg_ref, 2), wu1c_ref[...]) + bu1_ref[...])
        h = jnp.maximum(h, 0.0)
        h = jnp.maximum(dot(h, wu2_ref[...]) + bu2_ref[...], 0.0)
        u = dot(h, wu3_ref[...]) + bu3_ref[...]
        n = jnp.sqrt(jnp.sum(u * u, axis=1, keepdims=True))
        uout_ref[...] = u / jnp.maximum(n, 1e-12)
        h = (dot(half(iid_ref, 3), wi1a_ref[...])
             + dot(half(cat_ref, 4), wi1b_ref[...])
             + dot(te_ref[...], wi1c_ref[...]) + bi1_ref[...])
        h = jnp.maximum(h, 0.0)
        h = jnp.maximum(dot(h, wi2_ref[...]) + bi2_ref[...], 0.0)
        v = dot(h, wi3_ref[...]) + bi3_ref[...]
        n = jnp.sqrt(jnp.sum(v * v, axis=1, keepdims=True))
        iout_ref[...] = v / jnp.maximum(n, 1e-12)

    def batch_spec(d):
        return pl.BlockSpec((bm, d), lambda i: (i, 0))

    def full_spec(a):
        return pl.BlockSpec(a.shape, lambda i: (0,) * a.ndim)

    weights = (Wu1a, Wu1b, Wu1c, bu1, Wu2, bu2, Wu3, bu3,
               Wi1a, Wi1b, Wi1c, bi1, Wi2, bi2, Wi3, bi3)
    in_specs = ([batch_spec(PAIR)] * 5 + [batch_spec(8), batch_spec(EMB)]
                + [full_spec(w) for w in weights])
    out_specs = [batch_spec(EMB), batch_spec(EMB)]
    out_shape = [jax.ShapeDtypeStruct((B, EMB), f32)] * 2

    return pl.pallas_call(
        body, grid=grid, in_specs=in_specs, out_specs=out_specs,
        out_shape=out_shape,
    )(p_uid, p_age, p_reg, p_iid, p_cat, par, text_emb, *weights)


def kernel(user_id, user_age, user_region, item_id, item_category,
           text_features,
           E_user_id, E_user_age, E_user_region, E_item_id, E_item_category,
           Wt, bt, Wu1, bu1, Wu2, bu2, Wu3, bu3, Wi1, bi1, Wi2, bi2, Wi3, bi3):
    idxs = [user_id, user_age, user_region, item_id, item_category]
    tables2 = [t.reshape(-1, PAIR) for t in
               (E_user_id, E_user_age, E_user_region, E_item_id,
                E_item_category)]
    idx3 = [(i >> 1).reshape(NW, NCH, CH) for i in idxs]
    par = jnp.zeros((B, 8), jnp.float32)
    for k, i in enumerate(idxs):
        par = par.at[:, k].set((i & 1).astype(jnp.float32))

    pairs = _sc_gather5(tables2, idx3)
    text_emb = _tc_textproj(text_features, Wt, bt.reshape(1, EMB))
    user_out, item_out = _tc_towers(
        *pairs, par, text_emb,
        Wu1[0:EMB], Wu1[EMB:2 * EMB], Wu1[2 * EMB:3 * EMB],
        bu1.reshape(1, -1), Wu2, bu2.reshape(1, -1), Wu3, bu3.reshape(1, -1),
        Wi1[0:EMB], Wi1[EMB:2 * EMB], Wi1[2 * EMB:3 * EMB],
        bi1.reshape(1, -1), Wi2, bi2.reshape(1, -1), Wi3, bi3.reshape(1, -1))
    return (user_out, item_out)


# speedup vs baseline: 1.3777x; 1.3777x over previous
import functools

import jax
import jax.numpy as jnp
from jax import lax
from jax.experimental import pallas as pl
from jax.experimental.pallas import tpu as pltpu
from jax.experimental.pallas import tpu_sc as plsc

B = 16384
EMB = 64
PAIR = 2 * EMB
NC, NS = 2, 16
NW = NC * NS
BPW = B // NW
CH = 128
NCH = BPW // CH
NF = 5


def _fold_boundary(V):
    W = 1024
    return W * pl.cdiv(V, 2 * W)


def _tc_pack(table):
    V = table.shape[0]
    W = 1024
    hp = _fold_boundary(V)
    nb = hp // W

    vlast = pl.cdiv(V, W) - 1

    def body(a_ref, b_ref, o_ref):
        o_ref[...] = jnp.concatenate([a_ref[...].T, b_ref[...].T], axis=1)

    return pl.pallas_call(
        body, grid=(nb,),
        in_specs=[pl.BlockSpec((EMB, W), lambda i: (0, i)),
                  pl.BlockSpec((EMB, W),
                               lambda i: (0, jnp.minimum(i + nb, vlast)))],
        out_specs=pl.BlockSpec((W, PAIR), lambda i: (i, 0)),
        out_shape=jax.ShapeDtypeStruct((hp, PAIR), jnp.float32),
    )(table.T, table.T)


def _sc_gather5(tables2, idx3):
    mesh = plsc.VectorSubcoreMesh(core_axis_name="c", subcore_axis_name="s")
    out_type = tuple(
        jax.ShapeDtypeStruct((B, PAIR), jnp.float32) for _ in range(NF)
    )
    scratch = (
        [pltpu.VMEM((NCH, CH), jnp.int32) for _ in range(NF)]
        + [pltpu.VMEM((CH, PAIR), jnp.float32) for _ in range(NCH)]
        + [pltpu.SemaphoreType.DMA, pltpu.SemaphoreType.DMA]
    )

    @functools.partial(
        pl.kernel, out_type=out_type, mesh=mesh, scratch_types=scratch,
        compiler_params=pltpu.CompilerParams(use_tc_tiling_on_sc=True))
    def k(*refs):
        tbls = refs[0:NF]
        idxr = refs[NF:2 * NF]
        outs = refs[2 * NF:3 * NF]
        idx_v = refs[3 * NF:4 * NF]
        bufs = refs[4 * NF:4 * NF + NCH]
        gsem, wsem = refs[4 * NF + NCH], refs[4 * NF + NCH + 1]

        wid = lax.axis_index("s") * NC + lax.axis_index("c")
        base = wid * BPW

        for f in range(NF):
            pltpu.sync_copy(idxr[f].at[wid], idx_v[f])

        wb = [None] * NCH
        for f in range(NF):
            for j in range(NCH):
                if wb[j] is not None:
                    wb[j].wait()
            gh = [
                pltpu.async_copy(tbls[f].at[idx_v[f].at[j]], bufs[j], gsem)
                for j in range(NCH)
            ]
            for j in range(NCH):
                gh[j].wait()
                wb[j] = pltpu.async_copy(
                    bufs[j], outs[f].at[pl.ds(base + j * CH, CH)], wsem)
        for j in range(NCH):
            wb[j].wait()

    return k(*tables2, *idx3)


def _tc_textproj(text, Wt, bt):
    bm = 2048
    f32 = jnp.float32

    def body(text_ref, wt_ref, bt_ref, out_ref):
        out_ref[...] = lax.dot_general(
            text_ref[...], wt_ref[...], (((1,), (0,)), ((), ())),
            preferred_element_type=f32) + bt_ref[...]

    return pl.pallas_call(
        body, grid=(B // bm,),
        in_specs=[pl.BlockSpec((bm, text.shape[1]), lambda i: (i, 0)),
                  pl.BlockSpec(Wt.shape, lambda i: (0, 0)),
                  pl.BlockSpec(bt.shape, lambda i: (0, 0))],
        out_specs=pl.BlockSpec((bm, EMB), lambda i: (i, 0)),
        out_shape=jax.ShapeDtypeStruct((B, EMB), f32),
    )(text, Wt, bt)


def _tc_towers(p_uid, p_age, p_reg, p_iid, p_cat, par, text_emb,
               Wu1a, Wu1b, Wu1c, bu1, Wu2, bu2, Wu3, bu3,
               Wi1a, Wi1b, Wi1c, bi1, Wi2, bi2, Wi3, bi3):
    bm = 1024
    grid = (B // bm,)
    f32 = jnp.float32

    def dot(a, b):
        return lax.dot_general(a, b, (((1,), (0,)), ((), ())),
                               preferred_element_type=f32)

    def body(uid_ref, age_ref, reg_ref, iid_ref, cat_ref, par_ref, te_ref,
             wu1a_ref, wu1b_ref, wu1c_ref, bu1_ref, wu2_ref, bu2_ref,
             wu3_ref, bu3_ref,
             wi1a_ref, wi1b_ref, wi1c_ref, bi1_ref, wi2_ref, bi2_ref,
             wi3_ref, bi3_ref,
             uout_ref, iout_ref):
        par = par_ref[...]

        def half(pair_ref, k):
            p = par[:, k:k + 1] > 0.5
            return jnp.where(p, pair_ref[:, EMB:], pair_ref[:, :EMB])

        h = (dot(half(uid_ref, 0), wu1a_ref[...])
             + dot(half(age_ref, 1), wu1b_ref[...])
             + dot(half(reg_ref, 2), wu1c_ref[...]) + bu1_ref[...])
        h = jnp.maximum(h, 0.0)
        h = jnp.maximum(dot(h, wu2_ref[...]) + bu2_ref[...], 0.0)
        u = dot(h, wu3_ref[...]) + bu3_ref[...]
        n = jnp.sqrt(jnp.sum(u * u, axis=1, keepdims=True))
        uout_ref[...] = u / jnp.maximum(n, 1e-12)
        h = (dot(half(iid_ref, 3), wi1a_ref[...])
             + dot(half(cat_ref, 4), wi1b_ref[...])
             + dot(te_ref[...], wi1c_ref[...]) + bi1_ref[...])
        h = jnp.maximum(h, 0.0)
        h = jnp.maximum(dot(h, wi2_ref[...]) + bi2_ref[...], 0.0)
        v = dot(h, wi3_ref[...]) + bi3_ref[...]
        n = jnp.sqrt(jnp.sum(v * v, axis=1, keepdims=True))
        iout_ref[...] = v / jnp.maximum(n, 1e-12)

    def batch_spec(d):
        return pl.BlockSpec((bm, d), lambda i: (i, 0))

    def full_spec(a):
        return pl.BlockSpec(a.shape, lambda i: (0,) * a.ndim)

    weights = (Wu1a, Wu1b, Wu1c, bu1, Wu2, bu2, Wu3, bu3,
               Wi1a, Wi1b, Wi1c, bi1, Wi2, bi2, Wi3, bi3)
    in_specs = ([batch_spec(PAIR)] * 5 + [batch_spec(8), batch_spec(EMB)]
                + [full_spec(w) for w in weights])
    out_specs = [batch_spec(EMB), batch_spec(EMB)]
    out_shape = [jax.ShapeDtypeStruct((B, EMB), f32)] * 2

    return pl.pallas_call(
        body, grid=grid, in_specs=in_specs, out_specs=out_specs,
        out_shape=out_shape,
    )(p_uid, p_age, p_reg, p_iid, p_cat, par, text_emb, *weights)


def kernel(user_id, user_age, user_region, item_id, item_category,
           text_features,
           E_user_id, E_user_age, E_user_region, E_item_id, E_item_category,
           Wt, bt, Wu1, bu1, Wu2, bu2, Wu3, bu3, Wi1, bi1, Wi2, bi2, Wi3, bi3):
    idxs = [user_id, user_age, user_region, item_id, item_category]
    tabs = [E_user_id, E_user_age, E_user_region, E_item_id, E_item_category]
    fold = [True, False, False, True, False]
    tables2, pidxs, halves = [], [], []
    for t, i, fo in zip(tabs, idxs, fold):
        if fo:
            h = _fold_boundary(t.shape[0])
            tables2.append(_tc_pack(t))
            pidxs.append(jnp.where(i < h, i, i - h))
            halves.append((i >= h).astype(jnp.float32))
        else:
            tables2.append(t.reshape(-1, PAIR))
            pidxs.append(i >> 1)
            halves.append((i & 1).astype(jnp.float32))
    idx3 = [i.reshape(NW, NCH, CH) for i in pidxs]
    par = jnp.stack(halves + [jnp.zeros((B,), jnp.float32)] * 3, axis=1)

    pairs = _sc_gather5(tables2, idx3)
    text_emb = _tc_textproj(text_features, Wt, bt.reshape(1, EMB))
    user_out, item_out = _tc_towers(
        *pairs, par, text_emb,
        Wu1[0:EMB], Wu1[EMB:2 * EMB], Wu1[2 * EMB:3 * EMB],
        bu1.reshape(1, -1), Wu2, bu2.reshape(1, -1), Wu3, bu3.reshape(1, -1),
        Wi1[0:EMB], Wi1[EMB:2 * EMB], Wi1[2 * EMB:3 * EMB],
        bi1.reshape(1, -1), Wi2, bi2.reshape(1, -1), Wi3, bi3.reshape(1, -1))
    return (user_out, item_out)


# speedup vs baseline: 1.7214x; 1.2495x over previous
import functools

import jax
import jax.numpy as jnp
from jax import lax
from jax.experimental import pallas as pl
from jax.experimental.pallas import tpu as pltpu
from jax.experimental.pallas import tpu_sc as plsc

B = 16384
EMB = 64
PAIR = 2 * EMB
NC, NS = 2, 16
NW = NC * NS
BPW = B // NW
CH = 128
NCH = BPW // CH
NF = 5


def _fold_boundary(V):
    W = 2048
    return W * pl.cdiv(V, 2 * W)


def _tc_pack(table):
    V = table.shape[0]
    W = 2048
    hp = _fold_boundary(V)
    nb = hp // W

    vlast = pl.cdiv(V, W) - 1

    def body(a_ref, b_ref, o_ref):
        o_ref[...] = jnp.concatenate([a_ref[...].T, b_ref[...].T], axis=1)

    return pl.pallas_call(
        body, grid=(nb,),
        in_specs=[pl.BlockSpec((EMB, W), lambda i: (0, i)),
                  pl.BlockSpec((EMB, W),
                               lambda i: (0, jnp.minimum(i + nb, vlast)))],
        out_specs=pl.BlockSpec((W, PAIR), lambda i: (i, 0)),
        out_shape=jax.ShapeDtypeStruct((hp, PAIR), jnp.float32),
    )(table.T, table.T)


def _sc_gather5(tables2, idx3):
    mesh = plsc.VectorSubcoreMesh(core_axis_name="c", subcore_axis_name="s")
    out_type = tuple(
        jax.ShapeDtypeStruct((B, PAIR), jnp.float32) for _ in range(NF)
    )
    scratch = (
        [pltpu.VMEM((NCH, CH), jnp.int32) for _ in range(NF)]
        + [pltpu.VMEM((CH, PAIR), jnp.float32) for _ in range(NCH)]
        + [pltpu.SemaphoreType.DMA, pltpu.SemaphoreType.DMA]
    )

    @functools.partial(
        pl.kernel, out_type=out_type, mesh=mesh, scratch_types=scratch,
        compiler_params=pltpu.CompilerParams(use_tc_tiling_on_sc=True))
    def k(*refs):
        tbls = refs[0:NF]
        idxr = refs[NF:2 * NF]
        outs = refs[2 * NF:3 * NF]
        idx_v = refs[3 * NF:4 * NF]
        bufs = refs[4 * NF:4 * NF + NCH]
        gsem, wsem = refs[4 * NF + NCH], refs[4 * NF + NCH + 1]

        wid = lax.axis_index("s") * NC + lax.axis_index("c")
        base = wid * BPW

        for f in range(NF):
            pltpu.sync_copy(idxr[f].at[wid], idx_v[f])

        wb = [None] * NCH
        for f in range(NF):
            for j in range(NCH):
                if wb[j] is not None:
                    wb[j].wait()
            gh = [
                pltpu.async_copy(tbls[f].at[idx_v[f].at[j]], bufs[j], gsem)
                for j in range(NCH)
            ]
            for j in range(NCH):
                gh[j].wait()
                wb[j] = pltpu.async_copy(
                    bufs[j], outs[f].at[pl.ds(base + j * CH, CH)], wsem)
        for j in range(NCH):
            wb[j].wait()

    return k(*tables2, *idx3)


def _tc_textproj(text, Wt, bt):
    bm = 2048
    f32 = jnp.float32

    def body(text_ref, wt_ref, bt_ref, out_ref):
        out_ref[...] = lax.dot_general(
            text_ref[...], wt_ref[...], (((1,), (0,)), ((), ())),
            preferred_element_type=f32) + bt_ref[...]

    return pl.pallas_call(
        body, grid=(B // bm,),
        in_specs=[pl.BlockSpec((bm, text.shape[1]), lambda i: (i, 0)),
                  pl.BlockSpec(Wt.shape, lambda i: (0, 0)),
                  pl.BlockSpec(bt.shape, lambda i: (0, 0))],
        out_specs=pl.BlockSpec((bm, EMB), lambda i: (i, 0)),
        out_shape=jax.ShapeDtypeStruct((B, EMB), f32),
    )(text, Wt, bt)


def _tc_towers(p_uid, p_age, p_reg, p_iid, p_cat, par, text_emb,
               Wu1a, Wu1b, Wu1c, bu1, Wu2, bu2, Wu3, bu3,
               Wi1a, Wi1b, Wi1c, bi1, Wi2, bi2, Wi3, bi3):
    bm = 1024
    grid = (B // bm,)
    f32 = jnp.float32

    def dot(a, b):
        return lax.dot_general(a.astype(jnp.bfloat16), b.astype(jnp.bfloat16),
                               (((1,), (0,)), ((), ())),
                               preferred_element_type=f32)

    def body(uid_ref, age_ref, reg_ref, iid_ref, cat_ref, par_ref, te_ref,
             wu1a_ref, wu1b_ref, wu1c_ref, bu1_ref, wu2_ref, bu2_ref,
             wu3_ref, bu3_ref,
             wi1a_ref, wi1b_ref, wi1c_ref, bi1_ref, wi2_ref, bi2_ref,
             wi3_ref, bi3_ref,
             uout_ref, iout_ref):
        par = par_ref[...]

        def half(pair_ref, k):
            p = par[:, k:k + 1] > 0.5
            return jnp.where(p, pair_ref[:, EMB:], pair_ref[:, :EMB])

        h = (dot(half(uid_ref, 0), wu1a_ref[...])
             + dot(half(age_ref, 1), wu1b_ref[...])
             + dot(half(reg_ref, 2), wu1c_ref[...]) + bu1_ref[...])
        h = jnp.maximum(h, 0.0)
        h = jnp.maximum(dot(h, wu2_ref[...]) + bu2_ref[...], 0.0)
        u = dot(h, wu3_ref[...]) + bu3_ref[...]
        n = jnp.sqrt(jnp.sum(u * u, axis=1, keepdims=True))
        uout_ref[...] = u / jnp.maximum(n, 1e-12)
        h = (dot(half(iid_ref, 3), wi1a_ref[...])
             + dot(half(cat_ref, 4), wi1b_ref[...])
             + dot(te_ref[...], wi1c_ref[...]) + bi1_ref[...])
        h = jnp.maximum(h, 0.0)
        h = jnp.maximum(dot(h, wi2_ref[...]) + bi2_ref[...], 0.0)
        v = dot(h, wi3_ref[...]) + bi3_ref[...]
        n = jnp.sqrt(jnp.sum(v * v, axis=1, keepdims=True))
        iout_ref[...] = v / jnp.maximum(n, 1e-12)

    def batch_spec(d):
        return pl.BlockSpec((bm, d), lambda i: (i, 0))

    def full_spec(a):
        return pl.BlockSpec(a.shape, lambda i: (0,) * a.ndim)

    weights = (Wu1a, Wu1b, Wu1c, bu1, Wu2, bu2, Wu3, bu3,
               Wi1a, Wi1b, Wi1c, bi1, Wi2, bi2, Wi3, bi3)
    in_specs = ([batch_spec(PAIR)] * 5 + [batch_spec(8), batch_spec(EMB)]
                + [full_spec(w) for w in weights])
    out_specs = [batch_spec(EMB), batch_spec(EMB)]
    out_shape = [jax.ShapeDtypeStruct((B, EMB), f32)] * 2

    return pl.pallas_call(
        body, grid=grid, in_specs=in_specs, out_specs=out_specs,
        out_shape=out_shape,
    )(p_uid, p_age, p_reg, p_iid, p_cat, par, text_emb, *weights)


def kernel(user_id, user_age, user_region, item_id, item_category,
           text_features,
           E_user_id, E_user_age, E_user_region, E_item_id, E_item_category,
           Wt, bt, Wu1, bu1, Wu2, bu2, Wu3, bu3, Wi1, bi1, Wi2, bi2, Wi3, bi3):
    idxs = [user_id, user_age, user_region, item_id, item_category]
    tabs = [E_user_id, E_user_age, E_user_region, E_item_id, E_item_category]
    fold = [True, False, False, True, False]
    tables2, pidxs, halves = [], [], []
    for t, i, fo in zip(tabs, idxs, fold):
        if fo:
            h = _fold_boundary(t.shape[0])
            tables2.append(_tc_pack(t))
            pidxs.append(jnp.where(i < h, i, i - h))
            halves.append((i >= h).astype(jnp.float32))
        else:
            tables2.append(t.reshape(-1, PAIR))
            pidxs.append(i >> 1)
            halves.append((i & 1).astype(jnp.float32))
    idx3 = [i.reshape(NW, NCH, CH) for i in pidxs]
    par = jnp.stack(halves + [jnp.zeros((B,), jnp.float32)] * 3, axis=1)

    pairs = _sc_gather5(tables2, idx3)
    text_emb = _tc_textproj(text_features, Wt, bt.reshape(1, EMB))
    user_out, item_out = _tc_towers(
        *pairs, par, text_emb,
        Wu1[0:EMB], Wu1[EMB:2 * EMB], Wu1[2 * EMB:3 * EMB],
        bu1.reshape(1, -1), Wu2, bu2.reshape(1, -1), Wu3, bu3.reshape(1, -1),
        Wi1[0:EMB], Wi1[EMB:2 * EMB], Wi1[2 * EMB:3 * EMB],
        bi1.reshape(1, -1), Wi2, bi2.reshape(1, -1), Wi3, bi3.reshape(1, -1))
    return (user_out, item_out)


# speedup vs baseline: 2.0213x; 1.1742x over previous
import functools

import jax
import jax.numpy as jnp
from jax import lax
from jax.experimental import pallas as pl
from jax.experimental.pallas import tpu as pltpu
from jax.experimental.pallas import tpu_sc as plsc

B = 16384
EMB = 64
PAIR = 2 * EMB
NC, NS = 2, 16
NW = NC * NS
BPW = B // NW
CH = 128
NCH = BPW // CH
NF = 5


def _fold_boundary(V):
    W = 4096
    return W * pl.cdiv(V, 2 * W)


def _tc_pack(table):
    V = table.shape[0]
    W = 4096
    hp = _fold_boundary(V)
    nb = hp // W

    vlast = pl.cdiv(V, W) - 1

    def body(a_ref, b_ref, o_ref):
        o_ref[...] = jnp.concatenate([a_ref[...].T, b_ref[...].T], axis=1)

    return pl.pallas_call(
        body, grid=(nb,),
        in_specs=[pl.BlockSpec((EMB, W), lambda i: (0, i)),
                  pl.BlockSpec((EMB, W),
                               lambda i: (0, jnp.minimum(i + nb, vlast)))],
        out_specs=pl.BlockSpec((W, PAIR), lambda i: (i, 0)),
        out_shape=jax.ShapeDtypeStruct((hp, PAIR), jnp.float32),
    )(table.T, table.T)


def _sc_gather5(tables2, idx3):
    mesh = plsc.VectorSubcoreMesh(core_axis_name="c", subcore_axis_name="s")
    out_type = tuple(
        jax.ShapeDtypeStruct((B, PAIR), jnp.float32) for _ in range(NF)
    )
    scratch = (
        [pltpu.VMEM((NCH, CH), jnp.int32) for _ in range(NF)]
        + [pltpu.VMEM((CH, PAIR), jnp.float32) for _ in range(NCH)]
        + [pltpu.SemaphoreType.DMA, pltpu.SemaphoreType.DMA]
    )

    @functools.partial(
        pl.kernel, out_type=out_type, mesh=mesh, scratch_types=scratch,
        compiler_params=pltpu.CompilerParams(use_tc_tiling_on_sc=True))
    def k(*refs):
        tbls = refs[0:NF]
        idxr = refs[NF:2 * NF]
        outs = refs[2 * NF:3 * NF]
        idx_v = refs[3 * NF:4 * NF]
        bufs = refs[4 * NF:4 * NF + NCH]
        gsem, wsem = refs[4 * NF + NCH], refs[4 * NF + NCH + 1]

        wid = lax.axis_index("s") * NC + lax.axis_index("c")
        base = wid * BPW

        for f in range(NF):
            pltpu.sync_copy(idxr[f].at[wid], idx_v[f])

        wb = [None] * NCH
        for f in range(NF):
            for j in range(NCH):
                if wb[j] is not None:
                    wb[j].wait()
            gh = [
                pltpu.async_copy(tbls[f].at[idx_v[f].at[j]], bufs[j], gsem)
                for j in range(NCH)
            ]
            for j in range(NCH):
                gh[j].wait()
                wb[j] = pltpu.async_copy(
                    bufs[j], outs[f].at[pl.ds(base + j * CH, CH)], wsem)
        for j in range(NCH):
            wb[j].wait()

    return k(*tables2, *idx3)


def _tc_textproj(text, Wt, bt):
    bm = 2048
    f32 = jnp.float32

    def body(text_ref, wt_ref, bt_ref, out_ref):
        out_ref[...] = lax.dot_general(
            text_ref[...], wt_ref[...], (((1,), (0,)), ((), ())),
            preferred_element_type=f32) + bt_ref[...]

    return pl.pallas_call(
        body, grid=(B // bm,),
        in_specs=[pl.BlockSpec((bm, text.shape[1]), lambda i: (i, 0)),
                  pl.BlockSpec(Wt.shape, lambda i: (0, 0)),
                  pl.BlockSpec(bt.shape, lambda i: (0, 0))],
        out_specs=pl.BlockSpec((bm, EMB), lambda i: (i, 0)),
        out_shape=jax.ShapeDtypeStruct((B, EMB), f32),
    )(text, Wt, bt)


def _tc_towers(p_uid, p_age, p_reg, p_iid, p_cat, par, text_emb,
               Wu1a, Wu1b, Wu1c, bu1, Wu2, bu2, Wu3, bu3,
               Wi1a, Wi1b, Wi1c, bi1, Wi2, bi2, Wi3, bi3):
    bm = 2048
    grid = (B // bm,)
    f32 = jnp.float32

    def dot(a, b):
        return lax.dot_general(a.astype(jnp.bfloat16), b.astype(jnp.bfloat16),
                               (((1,), (0,)), ((), ())),
                               preferred_element_type=f32)

    def body(uid_ref, age_ref, reg_ref, iid_ref, cat_ref, par_ref, te_ref,
             wu1a_ref, wu1b_ref, wu1c_ref, bu1_ref, wu2_ref, bu2_ref,
             wu3_ref, bu3_ref,
             wi1a_ref, wi1b_ref, wi1c_ref, bi1_ref, wi2_ref, bi2_ref,
             wi3_ref, bi3_ref,
             uout_ref, iout_ref):
        par = par_ref[...]

        def half(pair_ref, k):
            p = par[:, k:k + 1] > 0.5
            return jnp.where(p, pair_ref[:, EMB:], pair_ref[:, :EMB])

        h = (dot(half(uid_ref, 0), wu1a_ref[...])
             + dot(half(age_ref, 1), wu1b_ref[...])
             + dot(half(reg_ref, 2), wu1c_ref[...]) + bu1_ref[...])
        h = jnp.maximum(h, 0.0)
        h = jnp.maximum(dot(h, wu2_ref[...]) + bu2_ref[...], 0.0)
        u = dot(h, wu3_ref[...]) + bu3_ref[...]
        n = jnp.sqrt(jnp.sum(u * u, axis=1, keepdims=True))
        uout_ref[...] = u / jnp.maximum(n, 1e-12)
        h = (dot(half(iid_ref, 3), wi1a_ref[...])
             + dot(half(cat_ref, 4), wi1b_ref[...])
             + dot(te_ref[...], wi1c_ref[...]) + bi1_ref[...])
        h = jnp.maximum(h, 0.0)
        h = jnp.maximum(dot(h, wi2_ref[...]) + bi2_ref[...], 0.0)
        v = dot(h, wi3_ref[...]) + bi3_ref[...]
        n = jnp.sqrt(jnp.sum(v * v, axis=1, keepdims=True))
        iout_ref[...] = v / jnp.maximum(n, 1e-12)

    def batch_spec(d):
        return pl.BlockSpec((bm, d), lambda i: (i, 0))

    def full_spec(a):
        return pl.BlockSpec(a.shape, lambda i: (0,) * a.ndim)

    weights = (Wu1a, Wu1b, Wu1c, bu1, Wu2, bu2, Wu3, bu3,
               Wi1a, Wi1b, Wi1c, bi1, Wi2, bi2, Wi3, bi3)
    in_specs = ([batch_spec(PAIR)] * 5 + [batch_spec(8), batch_spec(EMB)]
                + [full_spec(w) for w in weights])
    out_specs = [batch_spec(EMB), batch_spec(EMB)]
    out_shape = [jax.ShapeDtypeStruct((B, EMB), f32)] * 2

    return pl.pallas_call(
        body, grid=grid, in_specs=in_specs, out_specs=out_specs,
        out_shape=out_shape,
    )(p_uid, p_age, p_reg, p_iid, p_cat, par, text_emb, *weights)


def kernel(user_id, user_age, user_region, item_id, item_category,
           text_features,
           E_user_id, E_user_age, E_user_region, E_item_id, E_item_category,
           Wt, bt, Wu1, bu1, Wu2, bu2, Wu3, bu3, Wi1, bi1, Wi2, bi2, Wi3, bi3):
    idxs = [user_id, user_age, user_region, item_id, item_category]
    tabs = [E_user_id, E_user_age, E_user_region, E_item_id, E_item_category]
    fold = [True, False, False, True, False]
    tables2, pidxs, halves = [], [], []
    for t, i, fo in zip(tabs, idxs, fold):
        if fo:
            h = _fold_boundary(t.shape[0])
            tables2.append(_tc_pack(t))
            pidxs.append(jnp.where(i < h, i, i - h))
            halves.append((i >= h).astype(jnp.float32))
        else:
            tables2.append(t.reshape(-1, PAIR))
            pidxs.append(i >> 1)
            halves.append((i & 1).astype(jnp.float32))
    idx3 = [i.reshape(NW, NCH, CH) for i in pidxs]
    par = jnp.stack(halves + [jnp.zeros((B,), jnp.float32)] * 3, axis=1)

    pairs = _sc_gather5(tables2, idx3)
    text_emb = _tc_textproj(text_features, Wt, bt.reshape(1, EMB))
    user_out, item_out = _tc_towers(
        *pairs, par, text_emb,
        Wu1[0:EMB], Wu1[EMB:2 * EMB], Wu1[2 * EMB:3 * EMB],
        bu1.reshape(1, -1), Wu2, bu2.reshape(1, -1), Wu3, bu3.reshape(1, -1),
        Wi1[0:EMB], Wi1[EMB:2 * EMB], Wi1[2 * EMB:3 * EMB],
        bi1.reshape(1, -1), Wi2, bi2.reshape(1, -1), Wi3, bi3.reshape(1, -1))
    return (user_out, item_out)


# speedup vs baseline: 2.1615x; 1.0693x over previous
import functools

import jax
import jax.numpy as jnp
from jax import lax
from jax.experimental import pallas as pl
from jax.experimental.pallas import tpu as pltpu
from jax.experimental.pallas import tpu_sc as plsc

B = 16384
EMB = 64
PAIR = 2 * EMB
NC, NS = 2, 16
NW = NC * NS
BPW = B // NW
CH = 128
NCH = BPW // CH
NF = 5


def _fold_boundary(V):
    W = 8192
    return W * pl.cdiv(V, 2 * W)


def _tc_pack(table):
    V = table.shape[0]
    W = 8192
    hp = _fold_boundary(V)
    nb = hp // W

    vlast = pl.cdiv(V, W) - 1

    def body(a_ref, b_ref, o_ref):
        o_ref[...] = jnp.concatenate([a_ref[...].T, b_ref[...].T], axis=1)

    return pl.pallas_call(
        body, grid=(nb,),
        in_specs=[pl.BlockSpec((EMB, W), lambda i: (0, i)),
                  pl.BlockSpec((EMB, W),
                               lambda i: (0, jnp.minimum(i + nb, vlast)))],
        out_specs=pl.BlockSpec((W, PAIR), lambda i: (i, 0)),
        out_shape=jax.ShapeDtypeStruct((hp, PAIR), jnp.float32),
    )(table.T, table.T)


def _sc_gather5(tables2, idx3):
    mesh = plsc.VectorSubcoreMesh(core_axis_name="c", subcore_axis_name="s")
    out_type = tuple(
        jax.ShapeDtypeStruct((B, PAIR), jnp.float32) for _ in range(NF)
    )
    scratch = (
        [pltpu.VMEM((NCH, CH), jnp.int32) for _ in range(NF)]
        + [pltpu.VMEM((CH, PAIR), jnp.float32) for _ in range(NCH)]
        + [pltpu.SemaphoreType.DMA, pltpu.SemaphoreType.DMA]
    )

    @functools.partial(
        pl.kernel, out_type=out_type, mesh=mesh, scratch_types=scratch,
        compiler_params=pltpu.CompilerParams(use_tc_tiling_on_sc=True))
    def k(*refs):
        tbls = refs[0:NF]
        idxr = refs[NF:2 * NF]
        outs = refs[2 * NF:3 * NF]
        idx_v = refs[3 * NF:4 * NF]
        bufs = refs[4 * NF:4 * NF + NCH]
        gsem, wsem = refs[4 * NF + NCH], refs[4 * NF + NCH + 1]

        wid = lax.axis_index("s") * NC + lax.axis_index("c")
        base = wid * BPW

        for f in range(NF):
            pltpu.sync_copy(idxr[f].at[wid], idx_v[f])

        wb = [None] * NCH
        for f in range(NF):
            for j in range(NCH):
                if wb[j] is not None:
                    wb[j].wait()
            gh = [
                pltpu.async_copy(tbls[f].at[idx_v[f].at[j]], bufs[j], gsem)
                for j in range(NCH)
            ]
            for j in range(NCH):
                gh[j].wait()
                wb[j] = pltpu.async_copy(
                    bufs[j], outs[f].at[pl.ds(base + j * CH, CH)], wsem)
        for j in range(NCH):
            wb[j].wait()

    return k(*tables2, *idx3)


def _tc_textproj(text, Wt, bt):
    bm = 2048
    f32 = jnp.float32

    def body(text_ref, wt_ref, bt_ref, out_ref):
        out_ref[...] = lax.dot_general(
            text_ref[...], wt_ref[...], (((1,), (0,)), ((), ())),
            preferred_element_type=f32) + bt_ref[...]

    return pl.pallas_call(
        body, grid=(B // bm,),
        in_specs=[pl.BlockSpec((bm, text.shape[1]), lambda i: (i, 0)),
                  pl.BlockSpec(Wt.shape, lambda i: (0, 0)),
                  pl.BlockSpec(bt.shape, lambda i: (0, 0))],
        out_specs=pl.BlockSpec((bm, EMB), lambda i: (i, 0)),
        out_shape=jax.ShapeDtypeStruct((B, EMB), f32),
    )(text, Wt, bt)


def _tc_towers(p_uid, p_age, p_reg, p_iid, p_cat, par, text_emb,
               Wu1a, Wu1b, Wu1c, bu1, Wu2, bu2, Wu3, bu3,
               Wi1a, Wi1b, Wi1c, bi1, Wi2, bi2, Wi3, bi3):
    bm = 2048
    grid = (B // bm,)
    f32 = jnp.float32

    def dot(a, b):
        return lax.dot_general(a.astype(jnp.bfloat16), b.astype(jnp.bfloat16),
                               (((1,), (0,)), ((), ())),
                               preferred_element_type=f32)

    def body(uid_ref, age_ref, reg_ref, iid_ref, cat_ref, par_ref, te_ref,
             wu1a_ref, wu1b_ref, wu1c_ref, bu1_ref, wu2_ref, bu2_ref,
             wu3_ref, bu3_ref,
             wi1a_ref, wi1b_ref, wi1c_ref, bi1_ref, wi2_ref, bi2_ref,
             wi3_ref, bi3_ref,
             uout_ref, iout_ref):
        par = par_ref[...]

        def half(pair_ref, k):
            p = par[:, k:k + 1] > 0.5
            return jnp.where(p, pair_ref[:, EMB:], pair_ref[:, :EMB])

        h = (dot(half(uid_ref, 0), wu1a_ref[...])
             + dot(half(age_ref, 1), wu1b_ref[...])
             + dot(half(reg_ref, 2), wu1c_ref[...]) + bu1_ref[...])
        h = jnp.maximum(h, 0.0)
        h = jnp.maximum(dot(h, wu2_ref[...]) + bu2_ref[...], 0.0)
        u = dot(h, wu3_ref[...]) + bu3_ref[...]
        n = jnp.sqrt(jnp.sum(u * u, axis=1, keepdims=True))
        uout_ref[...] = u / jnp.maximum(n, 1e-12)
        h = (dot(half(iid_ref, 3), wi1a_ref[...])
             + dot(half(cat_ref, 4), wi1b_ref[...])
             + dot(te_ref[...], wi1c_ref[...]) + bi1_ref[...])
        h = jnp.maximum(h, 0.0)
        h = jnp.maximum(dot(h, wi2_ref[...]) + bi2_ref[...], 0.0)
        v = dot(h, wi3_ref[...]) + bi3_ref[...]
        n = jnp.sqrt(jnp.sum(v * v, axis=1, keepdims=True))
        iout_ref[...] = v / jnp.maximum(n, 1e-12)

    def batch_spec(d):
        return pl.BlockSpec((bm, d), lambda i: (i, 0))

    def full_spec(a):
        return pl.BlockSpec(a.shape, lambda i: (0,) * a.ndim)

    weights = (Wu1a, Wu1b, Wu1c, bu1, Wu2, bu2, Wu3, bu3,
               Wi1a, Wi1b, Wi1c, bi1, Wi2, bi2, Wi3, bi3)
    in_specs = ([batch_spec(PAIR)] * 5 + [batch_spec(8), batch_spec(EMB)]
                + [full_spec(w) for w in weights])
    out_specs = [batch_spec(EMB), batch_spec(EMB)]
    out_shape = [jax.ShapeDtypeStruct((B, EMB), f32)] * 2

    return pl.pallas_call(
        body, grid=grid, in_specs=in_specs, out_specs=out_specs,
        out_shape=out_shape,
    )(p_uid, p_age, p_reg, p_iid, p_cat, par, text_emb, *weights)


def kernel(user_id, user_age, user_region, item_id, item_category,
           text_features,
           E_user_id, E_user_age, E_user_region, E_item_id, E_item_category,
           Wt, bt, Wu1, bu1, Wu2, bu2, Wu3, bu3, Wi1, bi1, Wi2, bi2, Wi3, bi3):
    idxs = [user_id, user_age, user_region, item_id, item_category]
    tabs = [E_user_id, E_user_age, E_user_region, E_item_id, E_item_category]
    fold = [True, False, False, True, False]
    tables2, pidxs, halves = [], [], []
    for t, i, fo in zip(tabs, idxs, fold):
        if fo:
            h = _fold_boundary(t.shape[0])
            tables2.append(_tc_pack(t))
            pidxs.append(jnp.where(i < h, i, i - h))
            halves.append((i >= h).astype(jnp.float32))
        else:
            tables2.append(t.reshape(-1, PAIR))
            pidxs.append(i >> 1)
            halves.append((i & 1).astype(jnp.float32))
    idx3 = [i.reshape(NW, NCH, CH) for i in pidxs]
    par = jnp.stack(halves + [jnp.zeros((B,), jnp.float32)] * 3, axis=1)

    pairs = _sc_gather5(tables2, idx3)
    text_emb = _tc_textproj(text_features, Wt, bt.reshape(1, EMB))
    user_out, item_out = _tc_towers(
        *pairs, par, text_emb,
        Wu1[0:EMB], Wu1[EMB:2 * EMB], Wu1[2 * EMB:3 * EMB],
        bu1.reshape(1, -1), Wu2, bu2.reshape(1, -1), Wu3, bu3.reshape(1, -1),
        Wi1[0:EMB], Wi1[EMB:2 * EMB], Wi1[2 * EMB:3 * EMB],
        bi1.reshape(1, -1), Wi2, bi2.reshape(1, -1), Wi3, bi3.reshape(1, -1))
    return (user_out, item_out)
